# baseline (speedup 1.0000x reference)
import functools

import jax
import jax.numpy as jnp
from jax import lax
from jax.experimental import pallas as pl
from jax.experimental.pallas import tpu as pltpu
from jax.experimental.pallas import tpu_sc as plsc

NODES = 10000
T = 10
CH = 10
HID = 512
NA = 18
E = 160000
NTILES = 32
EPAD = 161792
EPT_DEG = EPAD // NTILES
EPT_SPMM = EPAD // 16
ECHUNK = EPT_SPMM // 2
UNROLL = 4
HF = CH // 2

_sc_mesh = plsc.VectorSubcoreMesh(core_axis_name="c", subcore_axis_name="s")
_sc_params = pltpu.CompilerParams(needs_layout_passes=False)


@functools.partial(
    pl.kernel,
    mesh=_sc_mesh,
    out_type=jax.ShapeDtypeStruct((NTILES * NODES,), jnp.float32),
    compiler_params=_sc_params,
    scratch_types=[
        pltpu.VMEM((EPT_DEG,), jnp.int32),
        pltpu.VMEM((EPT_DEG,), jnp.float32),
        pltpu.VMEM((NODES,), jnp.float32),
    ],
)
def _deg_kernel(dst_hbm, attr_hbm, zrow_hbm, out_hbm, dst_v, attr_v, deg_v):
    wid = lax.axis_index("s") * 2 + lax.axis_index("c")
    pltpu.sync_copy(dst_hbm.at[pl.ds(wid * EPT_DEG, EPT_DEG)], dst_v)
    pltpu.sync_copy(attr_hbm.at[pl.ds(wid * EPT_DEG, EPT_DEG)], attr_v)
    pltpu.sync_copy(zrow_hbm, deg_v)

    def ebody(i, carry):
        for g in range(UNROLL):
            base = (i * UNROLL + g) * 16
            d = dst_v[pl.ds(base, 16)]
            a = attr_v[pl.ds(base, 16)]
            plsc.addupdate_scatter(deg_v, [d], a)
        return carry

    lax.fori_loop(0, EPT_DEG // (16 * UNROLL), ebody, 0)
    pltpu.sync_copy(deg_v, out_hbm.at[pl.ds(wid * NODES, NODES)])


HFN = HF * NODES


@functools.partial(
    pl.kernel,
    mesh=_sc_mesh,
    out_type=jax.ShapeDtypeStruct((2 * 16 * 2 * HFN,), jnp.float32),
    compiler_params=_sc_params,
    scratch_types=[
        pltpu.VMEM((HFN,), jnp.float32),
        pltpu.VMEM((HFN,), jnp.float32),
        pltpu.VMEM((ECHUNK,), jnp.int32),
        pltpu.VMEM((ECHUNK,), jnp.int32),
        pltpu.VMEM((ECHUNK,), jnp.float32),
    ],
)
def _spmm_kernel(v_hbm, src_hbm, dst_hbm, attr_hbm, zhalf_hbm, out_hbm,
                 vh, acc, src_v, dst_v, attr_v):
    c = lax.axis_index("c")
    s = lax.axis_index("s")
    for h in range(2):
        pltpu.sync_copy(v_hbm.at[pl.ds(c * CH * NODES + h * HFN, HFN)], vh)
        pltpu.sync_copy(zhalf_hbm, acc)
        for chunk in range(2):
            ebase = s * EPT_SPMM + chunk * ECHUNK
            pltpu.sync_copy(src_hbm.at[pl.ds(ebase, ECHUNK)], src_v)
            pltpu.sync_copy(dst_hbm.at[pl.ds(ebase, ECHUNK)], dst_v)
            pltpu.sync_copy(attr_hbm.at[pl.ds(ebase, ECHUNK)], attr_v)

            def ebody(i, carry):
                for g in range(UNROLL):
                    base = (i * UNROLL + g) * 16
                    sidx = src_v[pl.ds(base, 16)]
                    didx = dst_v[pl.ds(base, 16)]
                    a = attr_v[pl.ds(base, 16)]
                    for f in range(HF):
                        col = plsc.load_gather(vh, [sidx + (f * NODES)])
                        plsc.addupdate_scatter(acc, [didx + (f * NODES)], col * a)
                return carry

            lax.fori_loop(0, ECHUNK // (16 * UNROLL), ebody, 0)
        obase = ((c * 16 + s) * 2 + h) * HFN
        pltpu.sync_copy(acc, out_hbm.at[pl.ds(obase, HFN)])


def _conv1_body(xT_ref, wr_ref, wi_ref, br_ref, bi_ref, yr_ref, yi_ref):
    xT = xT_ref[...]
    yr = jnp.dot(wr_ref[...], xT, preferred_element_type=jnp.float32) + br_ref[...]
    yi = jnp.dot(wi_ref[...], xT, preferred_element_type=jnp.float32) + bi_ref[...]
    yr_ref[...] = jnp.maximum(yr, 0.0)
    yi_ref[...] = jnp.maximum(yi, 0.0)


_conv1_call = pl.pallas_call(
    _conv1_body,
    out_shape=[
        jax.ShapeDtypeStruct((CH, NODES), jnp.float32),
        jax.ShapeDtypeStruct((CH, NODES), jnp.float32),
    ],
)


def _dinv_body(degp_ref, y_ref, v_ref, dinv_ref):
    deg = jnp.sum(degp_ref[...], axis=0, keepdims=True)
    dinv = jnp.where(deg > 0, lax.rsqrt(jnp.maximum(deg, 1e-12)), 0.0)
    dinv_ref[...] = dinv
    v_ref[...] = y_ref[...] * dinv[None]


_dinv_call = pl.pallas_call(
    _dinv_body,
    out_shape=[
        jax.ShapeDtypeStruct((2, CH, NODES), jnp.float32),
        jax.ShapeDtypeStruct((1, NODES), jnp.float32),
    ],
)


def _comb1_body(part_ref, y_ref, dinv_ref, t1_ref, v2_ref):
    sv = jnp.sum(part_ref[...], axis=1).reshape(2, CH, NODES)
    dinv = dinv_ref[...]
    t1 = y_ref[...] - sv * dinv[None]
    t1_ref[...] = t1
    v2_ref[...] = t1 * dinv[None]


_comb1_call = pl.pallas_call(
    _comb1_body,
    out_shape=[
        jax.ShapeDtypeStruct((2, CH, NODES), jnp.float32),
        jax.ShapeDtypeStruct((2, CH, NODES), jnp.float32),
    ],
)


def _comb2_body(part_ref, t1_ref, y_ref, dinv_ref, wt_ref, b_ref,
                zr_ref, zi_ref):
    st1 = jnp.sum(part_ref[...], axis=1).reshape(2, CH, NODES)
    dinv = dinv_ref[...]
    t1 = t1_ref[...]
    y = y_ref[...]
    t2 = 2.0 * (t1 - st1 * dinv[None]) - y
    wt = wt_ref[...]
    dot = lambda a, b: jnp.dot(a, b, preferred_element_type=jnp.float32)
    zr = dot(wt[0], y[0]) + dot(wt[1], t1[0]) + dot(wt[2], t2[0]) + b_ref[...]
    zi = dot(wt[0], y[1]) + dot(wt[1], t1[1]) + dot(wt[2], t2[1])
    zr_ref[...] = jnp.maximum(zr, 0.0)
    zi_ref[...] = jnp.maximum(zi, 0.0)


_comb2_call = pl.pallas_call(
    _comb2_body,
    out_shape=[
        jax.ShapeDtypeStruct((CH, NODES), jnp.float32),
        jax.ShapeDtypeStruct((CH, NODES), jnp.float32),
    ],
)


RB = 2000
NBLK = (NODES * T) // RB


def _fc1_body(z_ref, wr_ref, wi_ref, fb_ref, cw_ref, cb_ref, aw_ref, ab_ref,
              logits_ref, value_ref, accA, accC):
    k = pl.program_id(0)

    @pl.when(k == 0)
    def _init():
        accA[...] = jnp.zeros_like(accA)
        accC[...] = jnp.zeros_like(accC)

    z = z_ref[0]
    accA[...] += jnp.dot(z, wr_ref[...], preferred_element_type=jnp.float32)
    accC[...] += jnp.dot(z, wi_ref[...], preferred_element_type=jnp.float32)

    @pl.when(k == NBLK - 1)
    def _fin():
        A = accA[...]
        C = accC[...]
        fb = fb_ref[...]
        hr = jnp.maximum(A[0:1] - C[1:2] + fb[0:1], 0.0)
        hi = jnp.maximum(C[0:1] + A[1:2] + fb[1:2], 0.0)
        xcat = jnp.concatenate([hr, hi], axis=1)
        value_ref[...] = (
            jnp.dot(xcat, cw_ref[...], preferred_element_type=jnp.float32)
            + cb_ref[...]
        )
        logits_ref[...] = (
            jnp.dot(xcat, aw_ref[...], preferred_element_type=jnp.float32)
            + ab_ref[...]
        )


_fc1_call = pl.pallas_call(
    _fc1_body,
    grid=(NBLK,),
    in_specs=[
        pl.BlockSpec((1, 2, RB), lambda k: (k, 0, 0)),
        pl.BlockSpec((RB, HID), lambda k: (k, 0)),
        pl.BlockSpec((RB, HID), lambda k: (k, 0)),
        pl.BlockSpec((2, HID), lambda k: (0, 0)),
        pl.BlockSpec((2 * HID, 1), lambda k: (0, 0)),
        pl.BlockSpec((1, 1), lambda k: (0, 0)),
        pl.BlockSpec((2 * HID, NA), lambda k: (0, 0)),
        pl.BlockSpec((1, NA), lambda k: (0, 0)),
    ],
    out_specs=[
        pl.BlockSpec((1, NA), lambda k: (0, 0)),
        pl.BlockSpec((1, 1), lambda k: (0, 0)),
    ],
    out_shape=[
        jax.ShapeDtypeStruct((1, NA), jnp.float32),
        jax.ShapeDtypeStruct((1, 1), jnp.float32),
    ],
    scratch_shapes=[
        pltpu.VMEM((2, HID), jnp.float32),
        pltpu.VMEM((2, HID), jnp.float32),
    ],
)


def kernel(x, edge_index, edge_attr, conv1_Wr, conv1_Wi, conv1_br, conv1_bi,
           cheb_W, cheb_b, fc1_Wr, fc1_Wi, fc1_br, fc1_bi,
           critic_W, critic_b, actor_W, actor_b):
    xT = x.reshape(NODES, T).T
    yrT, yiT = _conv1_call(
        xT, conv1_Wr, conv1_Wi,
        conv1_br.reshape(CH, 1), conv1_bi.reshape(CH, 1),
    )
    y2 = jnp.stack([yrT, yiT])

    pad = EPAD - E
    src = jnp.concatenate([edge_index[0], jnp.zeros((pad,), jnp.int32)])
    dst = jnp.concatenate([edge_index[1], jnp.zeros((pad,), jnp.int32)])
    attr = jnp.concatenate([edge_attr, jnp.zeros((pad,), jnp.float32)])
    zrow = jnp.zeros((NODES,), jnp.float32)
    zhalf = jnp.zeros((HFN,), jnp.float32)
    pshape = (2, 16, 2, HF, NODES)

    deg_p = _deg_kernel(dst, attr, zrow).reshape(NTILES, NODES)
    v1, dinv = _dinv_call(deg_p, y2)

    part1 = _spmm_kernel(v1.reshape(-1), src, dst, attr, zhalf).reshape(pshape)
    t1, v2 = _comb1_call(part1, y2, dinv)

    part2 = _spmm_kernel(v2.reshape(-1), src, dst, attr, zhalf).reshape(pshape)
    wt = jnp.transpose(cheb_W, (0, 2, 1))
    zrT, ziT = _comb2_call(part2, t1, y2, dinv, wt, cheb_b.reshape(CH, 1))

    z2 = jnp.stack([zrT.T.reshape(-1), ziT.T.reshape(-1)])
    z3 = jnp.transpose(z2.reshape(2, NBLK, RB), (1, 0, 2))
    fb = jnp.stack([fc1_br, fc1_bi])
    logits, value = _fc1_call(
        z3, fc1_Wr, fc1_Wi, fb,
        critic_W, critic_b.reshape(1, 1), actor_W, actor_b.reshape(1, NA),
    )
    return (logits, value)


# speedup vs baseline: 1.1401x; 1.1401x over previous
import functools

import jax
import jax.numpy as jnp
from jax import lax
from jax.experimental import pallas as pl
from jax.experimental.pallas import tpu as pltpu
from jax.experimental.pallas import tpu_sc as plsc

NODES = 10000
T = 10
CH = 10
HID = 512
NA = 18
E = 160000
NTILES = 32
EPAD = 161792
EPT_DEG = EPAD // NTILES
EPT_SPMM = EPAD // 16
ECHUNK = EPT_SPMM // 2
UNROLL = 4
HF = CH // 2

_sc_mesh = plsc.VectorSubcoreMesh(core_axis_name="c", subcore_axis_name="s")
_sc_params = pltpu.CompilerParams(needs_layout_passes=False)


@functools.partial(
    pl.kernel,
    mesh=_sc_mesh,
    out_type=jax.ShapeDtypeStruct((NTILES * NODES,), jnp.float32),
    compiler_params=_sc_params,
    scratch_types=[
        pltpu.VMEM((EPT_DEG,), jnp.int32),
        pltpu.VMEM((EPT_DEG,), jnp.float32),
        pltpu.VMEM((NODES,), jnp.float32),
    ],
)
def _deg_kernel(dst_hbm, attr_hbm, zrow_hbm, out_hbm, dst_v, attr_v, deg_v):
    wid = lax.axis_index("s") * 2 + lax.axis_index("c")
    pltpu.sync_copy(dst_hbm.at[pl.ds(wid * EPT_DEG, EPT_DEG)], dst_v)
    pltpu.sync_copy(attr_hbm.at[pl.ds(wid * EPT_DEG, EPT_DEG)], attr_v)
    pltpu.sync_copy(zrow_hbm, deg_v)

    @plsc.parallel_loop(0, EPT_DEG // 16, 1, unroll=UNROLL)
    def _deg_loop(i):
        d = dst_v[pl.ds(i * 16, 16)]
        a = attr_v[pl.ds(i * 16, 16)]
        plsc.addupdate_scatter(deg_v, [d], a)
    pltpu.sync_copy(deg_v, out_hbm.at[pl.ds(wid * NODES, NODES)])


HFN = HF * NODES


@functools.partial(
    pl.kernel,
    mesh=_sc_mesh,
    out_type=jax.ShapeDtypeStruct((2 * 16 * 2 * HFN,), jnp.float32),
    compiler_params=_sc_params,
    scratch_types=[
        pltpu.VMEM((HFN,), jnp.float32),
        pltpu.VMEM((HFN,), jnp.float32),
        pltpu.VMEM((ECHUNK,), jnp.int32),
        pltpu.VMEM((ECHUNK,), jnp.int32),
        pltpu.VMEM((ECHUNK,), jnp.float32),
    ],
)
def _spmm_kernel(v_hbm, src_hbm, dst_hbm, attr_hbm, zhalf_hbm, out_hbm,
                 vh, acc, src_v, dst_v, attr_v):
    c = lax.axis_index("c")
    s = lax.axis_index("s")
    for h in range(2):
        pltpu.sync_copy(v_hbm.at[pl.ds(c * CH * NODES + h * HFN, HFN)], vh)
        pltpu.sync_copy(zhalf_hbm, acc)
        for chunk in range(2):
            ebase = s * EPT_SPMM + chunk * ECHUNK
            pltpu.sync_copy(src_hbm.at[pl.ds(ebase, ECHUNK)], src_v)
            pltpu.sync_copy(dst_hbm.at[pl.ds(ebase, ECHUNK)], dst_v)
            pltpu.sync_copy(attr_hbm.at[pl.ds(ebase, ECHUNK)], attr_v)

            @plsc.parallel_loop(0, ECHUNK // 16, 1, unroll=UNROLL)
            def _edge_loop(i):
                sidx = src_v[pl.ds(i * 16, 16)]
                didx = dst_v[pl.ds(i * 16, 16)]
                a = attr_v[pl.ds(i * 16, 16)]
                for f in range(HF):
                    col = plsc.load_gather(vh, [sidx + (f * NODES)])
                    plsc.addupdate_scatter(acc, [didx + (f * NODES)], col * a)
        obase = ((c * 16 + s) * 2 + h) * HFN
        pltpu.sync_copy(acc, out_hbm.at[pl.ds(obase, HFN)])


def _conv1_body(xT_ref, wr_ref, wi_ref, br_ref, bi_ref, yr_ref, yi_ref):
    xT = xT_ref[...]
    yr = jnp.dot(wr_ref[...], xT, preferred_element_type=jnp.float32) + br_ref[...]
    yi = jnp.dot(wi_ref[...], xT, preferred_element_type=jnp.float32) + bi_ref[...]
    yr_ref[...] = jnp.maximum(yr, 0.0)
    yi_ref[...] = jnp.maximum(yi, 0.0)


_conv1_call = pl.pallas_call(
    _conv1_body,
    out_shape=[
        jax.ShapeDtypeStruct((CH, NODES), jnp.float32),
        jax.ShapeDtypeStruct((CH, NODES), jnp.float32),
    ],
)


def _dinv_body(degp_ref, y_ref, v_ref, dinv_ref):
    deg = jnp.sum(degp_ref[...], axis=0, keepdims=True)
    dinv = jnp.where(deg > 0, lax.rsqrt(jnp.maximum(deg, 1e-12)), 0.0)
    dinv_ref[...] = dinv
    v_ref[...] = y_ref[...] * dinv[None]


_dinv_call = pl.pallas_call(
    _dinv_body,
    out_shape=[
        jax.ShapeDtypeStruct((2, CH, NODES), jnp.float32),
        jax.ShapeDtypeStruct((1, NODES), jnp.float32),
    ],
)


def _comb1_body(part_ref, y_ref, dinv_ref, t1_ref, v2_ref):
    sv = jnp.sum(part_ref[...], axis=1).reshape(2, CH, NODES)
    dinv = dinv_ref[...]
    t1 = y_ref[...] - sv * dinv[None]
    t1_ref[...] = t1
    v2_ref[...] = t1 * dinv[None]


_comb1_call = pl.pallas_call(
    _comb1_body,
    out_shape=[
        jax.ShapeDtypeStruct((2, CH, NODES), jnp.float32),
        jax.ShapeDtypeStruct((2, CH, NODES), jnp.float32),
    ],
)


def _comb2_body(part_ref, t1_ref, y_ref, dinv_ref, wt_ref, b_ref,
                zr_ref, zi_ref):
    st1 = jnp.sum(part_ref[...], axis=1).reshape(2, CH, NODES)
    dinv = dinv_ref[...]
    t1 = t1_ref[...]
    y = y_ref[...]
    t2 = 2.0 * (t1 - st1 * dinv[None]) - y
    wt = wt_ref[...]
    dot = lambda a, b: jnp.dot(a, b, preferred_element_type=jnp.float32)
    zr = dot(wt[0], y[0]) + dot(wt[1], t1[0]) + dot(wt[2], t2[0]) + b_ref[...]
    zi = dot(wt[0], y[1]) + dot(wt[1], t1[1]) + dot(wt[2], t2[1])
    zr_ref[...] = jnp.maximum(zr, 0.0)
    zi_ref[...] = jnp.maximum(zi, 0.0)


_comb2_call = pl.pallas_call(
    _comb2_body,
    out_shape=[
        jax.ShapeDtypeStruct((CH, NODES), jnp.float32),
        jax.ShapeDtypeStruct((CH, NODES), jnp.float32),
    ],
)


RB = 2000
NBLK = (NODES * T) // RB


def _fc1_body(z_ref, wr_ref, wi_ref, fb_ref, cw_ref, cb_ref, aw_ref, ab_ref,
              logits_ref, value_ref, accA, accC):
    k = pl.program_id(0)

    @pl.when(k == 0)
    def _init():
        accA[...] = jnp.zeros_like(accA)
        accC[...] = jnp.zeros_like(accC)

    z = z_ref[0]
    accA[...] += jnp.dot(z, wr_ref[...], preferred_element_type=jnp.float32)
    accC[...] += jnp.dot(z, wi_ref[...], preferred_element_type=jnp.float32)

    @pl.when(k == NBLK - 1)
    def _fin():
        A = accA[...]
        C = accC[...]
        fb = fb_ref[...]
        hr = jnp.maximum(A[0:1] - C[1:2] + fb[0:1], 0.0)
        hi = jnp.maximum(C[0:1] + A[1:2] + fb[1:2], 0.0)
        xcat = jnp.concatenate([hr, hi], axis=1)
        value_ref[...] = (
            jnp.dot(xcat, cw_ref[...], preferred_element_type=jnp.float32)
            + cb_ref[...]
        )
        logits_ref[...] = (
            jnp.dot(xcat, aw_ref[...], preferred_element_type=jnp.float32)
            + ab_ref[...]
        )


_fc1_call = pl.pallas_call(
    _fc1_body,
    grid=(NBLK,),
    in_specs=[
        pl.BlockSpec((1, 2, RB), lambda k: (k, 0, 0)),
        pl.BlockSpec((RB, HID), lambda k: (k, 0)),
        pl.BlockSpec((RB, HID), lambda k: (k, 0)),
        pl.BlockSpec((2, HID), lambda k: (0, 0)),
        pl.BlockSpec((2 * HID, 1), lambda k: (0, 0)),
        pl.BlockSpec((1, 1), lambda k: (0, 0)),
        pl.BlockSpec((2 * HID, NA), lambda k: (0, 0)),
        pl.BlockSpec((1, NA), lambda k: (0, 0)),
    ],
    out_specs=[
        pl.BlockSpec((1, NA), lambda k: (0, 0)),
        pl.BlockSpec((1, 1), lambda k: (0, 0)),
    ],
    out_shape=[
        jax.ShapeDtypeStruct((1, NA), jnp.float32),
        jax.ShapeDtypeStruct((1, 1), jnp.float32),
    ],
    scratch_shapes=[
        pltpu.VMEM((2, HID), jnp.float32),
        pltpu.VMEM((2, HID), jnp.float32),
    ],
)


def kernel(x, edge_index, edge_attr, conv1_Wr, conv1_Wi, conv1_br, conv1_bi,
           cheb_W, cheb_b, fc1_Wr, fc1_Wi, fc1_br, fc1_bi,
           critic_W, critic_b, actor_W, actor_b):
    xT = x.reshape(NODES, T).T
    yrT, yiT = _conv1_call(
        xT, conv1_Wr, conv1_Wi,
        conv1_br.reshape(CH, 1), conv1_bi.reshape(CH, 1),
    )
    y2 = jnp.stack([yrT, yiT])

    pad = EPAD - E
    src = jnp.concatenate([edge_index[0], jnp.zeros((pad,), jnp.int32)])
    dst = jnp.concatenate([edge_index[1], jnp.zeros((pad,), jnp.int32)])
    attr = jnp.concatenate([edge_attr, jnp.zeros((pad,), jnp.float32)])
    zrow = jnp.zeros((NODES,), jnp.float32)
    zhalf = jnp.zeros((HFN,), jnp.float32)
    pshape = (2, 16, 2, HF, NODES)

    deg_p = _deg_kernel(dst, attr, zrow).reshape(NTILES, NODES)
    v1, dinv = _dinv_call(deg_p, y2)

    part1 = _spmm_kernel(v1.reshape(-1), src, dst, attr, zhalf).reshape(pshape)
    t1, v2 = _comb1_call(part1, y2, dinv)

    part2 = _spmm_kernel(v2.reshape(-1), src, dst, attr, zhalf).reshape(pshape)
    wt = jnp.transpose(cheb_W, (0, 2, 1))
    zrT, ziT = _comb2_call(part2, t1, y2, dinv, wt, cheb_b.reshape(CH, 1))

    z2 = jnp.stack([zrT.T.reshape(-1), ziT.T.reshape(-1)])
    z3 = jnp.transpose(z2.reshape(2, NBLK, RB), (1, 0, 2))
    fb = jnp.stack([fc1_br, fc1_bi])
    logits, value = _fc1_call(
        z3, fc1_Wr, fc1_Wi, fb,
        critic_W, critic_b.reshape(1, 1), actor_W, actor_b.reshape(1, NA),
    )
    return (logits, value)
